# Initial kernel scaffold; baseline (speedup 1.0000x reference)
import jax
import jax.numpy as jnp
from jax.experimental import pallas as pl


def kernel(x, edge_index, W1, b1, W2, b2):
    raise NotImplementedError("write your pallas kernel here")


# speedup vs baseline: 10.6097x; 10.6097x over previous
import functools

import jax
import jax.numpy as jnp
from jax import lax
from jax.experimental import pallas as pl
from jax.experimental.pallas import tpu as pltpu
from jax.experimental.pallas import tpu_sc as plsc

N = 10000
NP = 10240
E = 160000
EP = 163840
D = 256
DH = 128
EB = 128
TPC = 16
BPT = EP // TPC // EB
RPT = NP // TPC
RB = 512
NBLK = NP // RB

_sc_mesh = plsc.VectorSubcoreMesh(core_axis_name="c", subcore_axis_name="s")


def _deg_body(srcp, dstp, zeros1, cs_out, cd_out, idx_v, ones_v, spc):
    c = lax.axis_index("c")
    s = lax.axis_index("s")
    for k in range(EB // 16):
        ones_v[pl.ds(k * 16, 16)] = jnp.ones((16,), jnp.float32)
    pltpu.sync_copy(zeros1.at[pl.ds(s * RPT, RPT)], spc.at[pl.ds(s * RPT, RPT)])
    plsc.subcore_barrier()

    def run(idx_hbm, out_hbm):
        pltpu.sync_copy(idx_hbm.at[pl.ds(s * BPT, BPT)], idx_v)

        def body(j, carry):
            pltpu.sync_copy(ones_v, spc.at[idx_v.at[j]], add=True)
            return carry

        lax.fori_loop(0, BPT, body, 0)
        plsc.subcore_barrier()
        pltpu.sync_copy(spc.at[pl.ds(s * RPT, RPT)], out_hbm.at[pl.ds(s * RPT, RPT)])

    @pl.when(c == 0)
    def _():
        run(srcp, cs_out)

    @pl.when(c == 1)
    def _():
        run(dstp, cd_out)


_deg_kernel = functools.partial(
    pl.kernel,
    out_type=(
        jax.ShapeDtypeStruct((NP,), jnp.float32),
        jax.ShapeDtypeStruct((NP,), jnp.float32),
    ),
    mesh=_sc_mesh,
    scratch_types=[
        pltpu.VMEM((BPT, EB), jnp.int32),
        pltpu.VMEM((EB,), jnp.float32),
        pltpu.VMEM_SHARED((NP,), jnp.float32),
    ],
)(_deg_body)


def _agg_body(hs_a, hs_b, srcp, dstp, zeros2, aa_out, ab_out,
              sidx, didx, buf, sem, spa):
    c = lax.axis_index("c")
    s = lax.axis_index("s")
    pltpu.sync_copy(srcp.at[pl.ds(s * BPT, BPT)], sidx)
    pltpu.sync_copy(dstp.at[pl.ds(s * BPT, BPT)], didx)
    pltpu.sync_copy(zeros2, spa.at[pl.ds(s * RPT, RPT)])
    plsc.subcore_barrier()

    def run(hs, out_hbm):
        def body(j, carry):
            pltpu.async_copy(hs.at[sidx.at[j]], buf, sem).wait()
            pltpu.sync_copy(buf, spa.at[didx.at[j]], add=True)
            return carry

        lax.fori_loop(0, BPT, body, 0)
        plsc.subcore_barrier()
        pltpu.sync_copy(spa.at[pl.ds(s * RPT, RPT)], out_hbm.at[pl.ds(s * RPT, RPT)])

    @pl.when(c == 0)
    def _():
        run(hs_a, aa_out)

    @pl.when(c == 1)
    def _():
        run(hs_b, ab_out)


_agg_kernel = functools.partial(
    pl.kernel,
    out_type=(
        jax.ShapeDtypeStruct((NP, DH), jnp.float32),
        jax.ShapeDtypeStruct((NP, DH), jnp.float32),
    ),
    mesh=_sc_mesh,
    scratch_types=[
        pltpu.VMEM((BPT, EB), jnp.int32),
        pltpu.VMEM((BPT, EB), jnp.int32),
        pltpu.VMEM((EB, DH), jnp.float32),
        pltpu.SemaphoreType.DMA,
        pltpu.VMEM_SHARED((NP, DH), jnp.float32),
    ],
)(_agg_body)


def _mm1_body(x_ref, w_ref, cs_ref, oa_ref, ob_ref):
    h = jnp.dot(x_ref[...], w_ref[...], preferred_element_type=jnp.float32)
    hs = h * lax.rsqrt(cs_ref[...] + 1.0)
    oa_ref[...] = hs[:, :DH]
    ob_ref[...] = hs[:, DH:]


def _mid_body(aa_ref, ab_ref, ha_ref, hb_ref, cd_ref, cs_ref, b1_ref, w2_ref,
              oa_ref, ob_ref):
    nd = lax.rsqrt(cd_ref[...] + 1.0)
    ns = lax.rsqrt(cs_ref[...] + 1.0)
    h1 = jnp.concatenate(
        [aa_ref[...] + ha_ref[...], ab_ref[...] + hb_ref[...]], axis=1)
    h1 = jnp.maximum(h1 * nd + b1_ref[...], 0.0)
    h2 = jnp.dot(h1, w2_ref[...], preferred_element_type=jnp.float32) * ns
    oa_ref[...] = h2[:, :DH]
    ob_ref[...] = h2[:, DH:]


def _out_body(aa_ref, ab_ref, ha_ref, hb_ref, cd_ref, b2_ref, o_ref):
    nd = lax.rsqrt(cd_ref[...] + 1.0)
    h = jnp.concatenate(
        [aa_ref[...] + ha_ref[...], ab_ref[...] + hb_ref[...]], axis=1)
    o_ref[...] = jax.nn.sigmoid(h * nd + b2_ref[...])


_row_spec = pl.BlockSpec((RB, D), lambda i: (i, 0))
_half_spec = pl.BlockSpec((RB, DH), lambda i: (i, 0))
_cnt_spec = pl.BlockSpec((RB, 1), lambda i: (i, 0))
_w_spec = pl.BlockSpec((D, D), lambda i: (0, 0))
_b_spec = pl.BlockSpec((1, D), lambda i: (0, 0))

_mm1_kernel = pl.pallas_call(
    _mm1_body,
    grid=(NBLK,),
    in_specs=[_row_spec, _w_spec, _cnt_spec],
    out_specs=(_half_spec, _half_spec),
    out_shape=(
        jax.ShapeDtypeStruct((NP, DH), jnp.float32),
        jax.ShapeDtypeStruct((NP, DH), jnp.float32),
    ),
)

_mid_kernel = pl.pallas_call(
    _mid_body,
    grid=(NBLK,),
    in_specs=[_half_spec, _half_spec, _half_spec, _half_spec,
              _cnt_spec, _cnt_spec, _b_spec, _w_spec],
    out_specs=(_half_spec, _half_spec),
    out_shape=(
        jax.ShapeDtypeStruct((NP, DH), jnp.float32),
        jax.ShapeDtypeStruct((NP, DH), jnp.float32),
    ),
)

_out_kernel = pl.pallas_call(
    _out_body,
    grid=(NBLK,),
    in_specs=[_half_spec, _half_spec, _half_spec, _half_spec,
              _cnt_spec, _b_spec],
    out_specs=_row_spec,
    out_shape=jax.ShapeDtypeStruct((NP, D), jnp.float32),
)


def kernel(x, edge_index, W1, b1, W2, b2):
    src = edge_index[0].astype(jnp.int32)
    dst = edge_index[1].astype(jnp.int32)
    pad = N + (jnp.arange(EP - E, dtype=jnp.int32) % (NP - N))
    srcp = jnp.concatenate([src, pad]).reshape(EP // EB, EB)
    dstp = jnp.concatenate([dst, pad]).reshape(EP // EB, EB)
    xp = jnp.pad(x, ((0, NP - N), (0, 0)))
    zeros1 = jnp.zeros((NP,), jnp.float32)
    zeros2 = jnp.zeros((RPT, DH), jnp.float32)

    cs, cd = _deg_kernel(srcp, dstp, zeros1)
    cs2 = cs.reshape(NP, 1)
    cd2 = cd.reshape(NP, 1)

    ha, hb = _mm1_kernel(xp, W1, cs2)
    aa, ab = _agg_kernel(ha, hb, srcp, dstp, zeros2)
    ha2, hb2 = _mid_kernel(aa, ab, ha, hb, cd2, cs2, b1.reshape(1, D), W2)
    aa2, ab2 = _agg_kernel(ha2, hb2, srcp, dstp, zeros2)
    out = _out_kernel(aa2, ab2, ha2, hb2, cd2, b2.reshape(1, D))
    return out[:N]


# speedup vs baseline: 13.7882x; 1.2996x over previous
import functools

import jax
import jax.numpy as jnp
from jax import lax
from jax.experimental import pallas as pl
from jax.experimental.pallas import tpu as pltpu
from jax.experimental.pallas import tpu_sc as plsc

N = 10000
NP = 10240
E = 160000
EP = 163840
D = 256
DH = 128
EB = 80
TPC = 16
BPT = EP // TPC // EB
RPT = NP // TPC
RB = 512
NBLK = NP // RB

_sc_mesh = plsc.VectorSubcoreMesh(core_axis_name="c", subcore_axis_name="s")


def _deg_body(srcp, dstp, zeros1, cs_out, cd_out, idx_v, ones_v, spc):
    c = lax.axis_index("c")
    s = lax.axis_index("s")
    for k in range(EB // 16):
        ones_v[pl.ds(k * 16, 16)] = jnp.ones((16,), jnp.float32)
    pltpu.sync_copy(zeros1.at[pl.ds(s * RPT, RPT)], spc.at[pl.ds(s * RPT, RPT)])
    plsc.subcore_barrier()

    def run(idx_hbm, out_hbm):
        pltpu.sync_copy(idx_hbm.at[pl.ds(s * BPT, BPT)], idx_v)

        def body(j, carry):
            pltpu.sync_copy(ones_v, spc.at[idx_v.at[j]], add=True)
            return carry

        lax.fori_loop(0, BPT, body, 0)
        plsc.subcore_barrier()
        pltpu.sync_copy(spc.at[pl.ds(s * RPT, RPT)], out_hbm.at[pl.ds(s * RPT, RPT)])

    @pl.when(c == 0)
    def _():
        run(srcp, cs_out)

    @pl.when(c == 1)
    def _():
        run(dstp, cd_out)


_deg_kernel = functools.partial(
    pl.kernel,
    out_type=(
        jax.ShapeDtypeStruct((NP,), jnp.float32),
        jax.ShapeDtypeStruct((NP,), jnp.float32),
    ),
    mesh=_sc_mesh,
    scratch_types=[
        pltpu.VMEM((BPT, EB), jnp.int32),
        pltpu.VMEM((EB,), jnp.float32),
        pltpu.VMEM_SHARED((NP,), jnp.float32),
    ],
)(_deg_body)


NBUF = 2


def _agg_body(hs_a, hs_b, srcp, dstp, zeros2, aa_out, ab_out,
              sidx, didx, buf0, buf1, gs0, gs1, spa):
    bufs = [buf0, buf1]
    gsems = [gs0, gs1]
    c = lax.axis_index("c")
    s = lax.axis_index("s")
    pltpu.sync_copy(srcp.at[pl.ds(s * BPT, BPT)], sidx)
    pltpu.sync_copy(dstp.at[pl.ds(s * BPT, BPT)], didx)
    pltpu.sync_copy(zeros2, spa.at[pl.ds(s * RPT, RPT)])
    plsc.subcore_barrier()

    def run(hs, out_hbm):
        for k in range(NBUF):
            pltpu.async_copy(hs.at[sidx.at[k]], bufs[k], gsems[k])

        def body(j, carry):
            for k in range(NBUF):
                e = j * NBUF + k
                pltpu.make_async_copy(hs.at[sidx.at[e]], bufs[k], gsems[k]).wait()
                pltpu.sync_copy(bufs[k], spa.at[didx.at[e]], add=True)

                @pl.when(e < BPT - NBUF)
                def _():
                    pltpu.async_copy(hs.at[sidx.at[e + NBUF]], bufs[k], gsems[k])

            return carry

        lax.fori_loop(0, BPT // NBUF, body, 0)
        plsc.subcore_barrier()
        pltpu.sync_copy(spa.at[pl.ds(s * RPT, RPT)], out_hbm.at[pl.ds(s * RPT, RPT)])

    @pl.when(c == 0)
    def _():
        run(hs_a, aa_out)

    @pl.when(c == 1)
    def _():
        run(hs_b, ab_out)


_agg_kernel = functools.partial(
    pl.kernel,
    out_type=(
        jax.ShapeDtypeStruct((NP, DH), jnp.float32),
        jax.ShapeDtypeStruct((NP, DH), jnp.float32),
    ),
    mesh=_sc_mesh,
    compiler_params=pltpu.CompilerParams(use_tc_tiling_on_sc=False),
    scratch_types=[
        pltpu.VMEM((BPT, EB), jnp.int32),
        pltpu.VMEM((BPT, EB), jnp.int32),
        pltpu.VMEM((EB, DH), jnp.float32),
        pltpu.VMEM((EB, DH), jnp.float32),
        pltpu.SemaphoreType.DMA,
        pltpu.SemaphoreType.DMA,
        pltpu.VMEM_SHARED((NP, DH), jnp.float32),
    ],
)(_agg_body)


def _mm1_body(x_ref, w_ref, cs_ref, oa_ref, ob_ref):
    h = jnp.dot(x_ref[...], w_ref[...], preferred_element_type=jnp.float32)
    hs = h * lax.rsqrt(cs_ref[...] + 1.0)
    oa_ref[...] = hs[:, :DH]
    ob_ref[...] = hs[:, DH:]


def _mid_body(aa_ref, ab_ref, ha_ref, hb_ref, cd_ref, cs_ref, b1_ref, w2_ref,
              oa_ref, ob_ref):
    nd = lax.rsqrt(cd_ref[...] + 1.0)
    ns = lax.rsqrt(cs_ref[...] + 1.0)
    h1 = jnp.concatenate(
        [aa_ref[...] + ha_ref[...], ab_ref[...] + hb_ref[...]], axis=1)
    h1 = jnp.maximum(h1 * nd + b1_ref[...], 0.0)
    h2 = jnp.dot(h1, w2_ref[...], preferred_element_type=jnp.float32) * ns
    oa_ref[...] = h2[:, :DH]
    ob_ref[...] = h2[:, DH:]


def _out_body(aa_ref, ab_ref, ha_ref, hb_ref, cd_ref, b2_ref, o_ref):
    nd = lax.rsqrt(cd_ref[...] + 1.0)
    h = jnp.concatenate(
        [aa_ref[...] + ha_ref[...], ab_ref[...] + hb_ref[...]], axis=1)
    o_ref[...] = jax.nn.sigmoid(h * nd + b2_ref[...])


_row_spec = pl.BlockSpec((RB, D), lambda i: (i, 0))
_half_spec = pl.BlockSpec((RB, DH), lambda i: (i, 0))
_cnt_spec = pl.BlockSpec((RB, 1), lambda i: (i, 0))
_w_spec = pl.BlockSpec((D, D), lambda i: (0, 0))
_b_spec = pl.BlockSpec((1, D), lambda i: (0, 0))

_mm1_kernel = pl.pallas_call(
    _mm1_body,
    grid=(NBLK,),
    in_specs=[_row_spec, _w_spec, _cnt_spec],
    out_specs=(_half_spec, _half_spec),
    out_shape=(
        jax.ShapeDtypeStruct((NP, DH), jnp.float32),
        jax.ShapeDtypeStruct((NP, DH), jnp.float32),
    ),
)

_mid_kernel = pl.pallas_call(
    _mid_body,
    grid=(NBLK,),
    in_specs=[_half_spec, _half_spec, _half_spec, _half_spec,
              _cnt_spec, _cnt_spec, _b_spec, _w_spec],
    out_specs=(_half_spec, _half_spec),
    out_shape=(
        jax.ShapeDtypeStruct((NP, DH), jnp.float32),
        jax.ShapeDtypeStruct((NP, DH), jnp.float32),
    ),
)

_out_kernel = pl.pallas_call(
    _out_body,
    grid=(NBLK,),
    in_specs=[_half_spec, _half_spec, _half_spec, _half_spec,
              _cnt_spec, _b_spec],
    out_specs=_row_spec,
    out_shape=jax.ShapeDtypeStruct((NP, D), jnp.float32),
)


def kernel(x, edge_index, W1, b1, W2, b2):
    src = edge_index[0].astype(jnp.int32)
    dst = edge_index[1].astype(jnp.int32)
    pad = N + (jnp.arange(EP - E, dtype=jnp.int32) % (NP - N))
    srcp = jnp.concatenate([src, pad]).reshape(EP // EB, EB)
    dstp = jnp.concatenate([dst, pad]).reshape(EP // EB, EB)
    xp = jnp.pad(x, ((0, NP - N), (0, 0)))
    zeros1 = jnp.zeros((NP,), jnp.float32)
    zeros2 = jnp.zeros((RPT, DH), jnp.float32)

    cs, cd = _deg_kernel(srcp, dstp, zeros1)
    cs2 = cs.reshape(NP, 1)
    cd2 = cd.reshape(NP, 1)

    ha, hb = _mm1_kernel(xp, W1, cs2)
    aa, ab = _agg_kernel(ha, hb, srcp, dstp, zeros2)
    ha2, hb2 = _mid_kernel(aa, ab, ha, hb, cd2, cs2, b1.reshape(1, D), W2)
    aa2, ab2 = _agg_kernel(ha2, hb2, srcp, dstp, zeros2)
    out = _out_kernel(aa2, ab2, ha2, hb2, cd2, b2.reshape(1, D))
    return out[:N]
